# Initial kernel scaffold; baseline (speedup 1.0000x reference)
import jax
import jax.numpy as jnp
from jax.experimental import pallas as pl


def kernel(x, tables):
    raise NotImplementedError("write your pallas kernel here")


# speedup vs baseline: 6.6891x; 6.6891x over previous
import functools

import jax
import jax.numpy as jnp
from jax import lax
from jax.experimental import pallas as pl
from jax.experimental.pallas import tpu as pltpu
from jax.experimental.pallas import tpu_sc as plsc

NUM_INPUTS = 4
NUM_EMBEDDINGS = 100000
DIM = 64
LANES = 16
NUM_CORES = 2
NUM_SUBCORES = 16
NW = NUM_CORES * NUM_SUBCORES
R = 128


@functools.lru_cache(maxsize=None)
def _build(rows: int):
    assert rows % (NW * R) == 0
    per_w = rows // NW
    n_chunks = per_w // R
    mesh = plsc.VectorSubcoreMesh(
        core_axis_name="c", subcore_axis_name="s",
        num_cores=NUM_CORES, num_subcores=NUM_SUBCORES)

    @functools.partial(
        pl.kernel,
        out_type=jax.ShapeDtypeStruct((rows, DIM), jnp.float32),
        mesh=mesh,
        scratch_types=[
            pltpu.VMEM((NUM_INPUTS, R), jnp.int32),
            pltpu.VMEM((NUM_INPUTS, R, DIM), jnp.float32),
            pltpu.VMEM((R, DIM), jnp.float32),
            pltpu.SemaphoreType.DMA,
        ],
        compiler_params=pltpu.CompilerParams(use_tc_tiling_on_sc=False),
    )
    def mimo(xf_hbm, tab_hbm, out_hbm, idx_v, g_v, o_v, gsem):
        wid = lax.axis_index("s") * NUM_CORES + lax.axis_index("c")
        w_base = wid * per_w

        def chunk_body(ci, _):
            base = w_base + ci * R
            for i in range(NUM_INPUTS):
                pltpu.sync_copy(xf_hbm.at[i, pl.ds(base, R)], idx_v.at[i])
            for i in range(1, NUM_INPUTS):
                for j in range(R // LANES):
                    sl = pl.ds(j * LANES, LANES)
                    idx_v[i, sl] = idx_v[i, sl] + (i * NUM_EMBEDDINGS)
            dmas = [
                pltpu.async_copy(tab_hbm.at[idx_v.at[i]], g_v.at[i], gsem)
                for i in range(NUM_INPUTS)
            ]
            for d in dmas:
                d.wait()

            def acc_body(r, _):
                for j in range(DIM // LANES):
                    sl = pl.ds(j * LANES, LANES)
                    o_v[r, sl] = (g_v[0, r, sl] + g_v[1, r, sl]) + (
                        g_v[2, r, sl] + g_v[3, r, sl])
                return 0
            lax.fori_loop(0, R, acc_body, 0)
            pltpu.sync_copy(o_v, out_hbm.at[pl.ds(base, R)])
            return 0

        lax.fori_loop(0, n_chunks, chunk_body, 0)

    return mimo


def kernel(x, tables):
    num, vocab, dim = tables.shape
    seq = x.shape[-1]
    batch = x.shape[0] // num
    rows = batch * seq
    xf = x.reshape(num, rows)
    tf = tables.reshape(num * vocab, dim)
    out = _build(rows)(xf, tf)
    return out.reshape(batch, seq, dim)


# speedup vs baseline: 6.9206x; 1.0346x over previous
import functools

import jax
import jax.numpy as jnp
from jax import lax
from jax.experimental import pallas as pl
from jax.experimental.pallas import tpu as pltpu
from jax.experimental.pallas import tpu_sc as plsc

NUM_INPUTS = 4
NUM_EMBEDDINGS = 100000
DIM = 64
LANES = 16
NUM_CORES = 2
NUM_SUBCORES = 16
NW = NUM_CORES * NUM_SUBCORES
R = 128


@functools.lru_cache(maxsize=None)
def _build(rows: int):
    assert rows % (NW * R) == 0
    per_w = rows // NW
    n_chunks = per_w // R
    mesh = plsc.VectorSubcoreMesh(
        core_axis_name="c", subcore_axis_name="s",
        num_cores=NUM_CORES, num_subcores=NUM_SUBCORES)

    @functools.partial(
        pl.kernel,
        out_type=jax.ShapeDtypeStruct((rows, DIM), jnp.float32),
        mesh=mesh,
        scratch_types=[
            pltpu.VMEM((NUM_INPUTS, R), jnp.int32),
            pltpu.VMEM((R, DIM), jnp.float32),
            pltpu.SemaphoreType.DMA,
        ],
        compiler_params=pltpu.CompilerParams(use_tc_tiling_on_sc=False),
    )
    def mimo(xf_hbm, tab_hbm, out_hbm, idx_v, o_v, gsem):
        wid = lax.axis_index("s") * NUM_CORES + lax.axis_index("c")
        w_base = wid * per_w

        def chunk_body(ci, _):
            base = w_base + ci * R
            for i in range(NUM_INPUTS):
                pltpu.sync_copy(xf_hbm.at[i, pl.ds(base, R)], idx_v.at[i])
            for i in range(1, NUM_INPUTS):
                for j in range(R // LANES):
                    sl = pl.ds(j * LANES, LANES)
                    idx_v[i, sl] = idx_v[i, sl] + (i * NUM_EMBEDDINGS)
            pltpu.async_copy(tab_hbm.at[idx_v.at[0]], o_v, gsem).wait()
            dmas = [
                pltpu.async_copy(tab_hbm.at[idx_v.at[i]], o_v, gsem, add=True)
                for i in range(1, NUM_INPUTS)
            ]
            for d in dmas:
                d.wait()
            pltpu.sync_copy(o_v, out_hbm.at[pl.ds(base, R)])
            return 0

        lax.fori_loop(0, n_chunks, chunk_body, 0)

    return mimo


def kernel(x, tables):
    num, vocab, dim = tables.shape
    seq = x.shape[-1]
    batch = x.shape[0] // num
    rows = batch * seq
    xf = x.reshape(num, rows)
    tf = tables.reshape(num * vocab, dim)
    out = _build(rows)(xf, tf)
    return out.reshape(batch, seq, dim)


# speedup vs baseline: 9.2273x; 1.3333x over previous
import functools

import jax
import jax.numpy as jnp
from jax import lax
from jax.experimental import pallas as pl
from jax.experimental.pallas import tpu as pltpu
from jax.experimental.pallas import tpu_sc as plsc

NUM_INPUTS = 4
NUM_EMBEDDINGS = 100000
DIM = 64
LANES = 16
NUM_CORES = 2
NUM_SUBCORES = 16
NW = NUM_CORES * NUM_SUBCORES
R = 128
NB = 3


@functools.lru_cache(maxsize=None)
def _build(rows: int):
    assert rows % (NW * R) == 0
    per_w = rows // NW
    n_chunks = per_w // R
    mesh = plsc.VectorSubcoreMesh(
        core_axis_name="c", subcore_axis_name="s",
        num_cores=NUM_CORES, num_subcores=NUM_SUBCORES)

    @functools.partial(
        pl.kernel,
        out_type=jax.ShapeDtypeStruct((rows, DIM), jnp.float32),
        mesh=mesh,
        scratch_types=[
            pltpu.VMEM((NUM_INPUTS, per_w), jnp.int32),
            pltpu.VMEM((NB, R, DIM), jnp.float32),
            pltpu.SemaphoreType.DMA,
            pltpu.SemaphoreType.DMA((NB,)),
            pltpu.SemaphoreType.DMA((NB,)),
            pltpu.SemaphoreType.DMA((NB,)),
        ],
        compiler_params=pltpu.CompilerParams(use_tc_tiling_on_sc=False),
    )
    def mimo(xf_hbm, tab_hbm, out_hbm, idx_v, o_v, isem, g0sem, gasem, osem):
        wid = lax.axis_index("s") * NUM_CORES + lax.axis_index("c")
        w_base = wid * per_w

        pltpu.async_copy(
            xf_hbm.at[:, pl.ds(w_base, per_w)], idx_v, isem).wait()

        def bias_body(j, _):
            sl = pl.ds(j * LANES, LANES)
            for i in range(1, NUM_INPUTS):
                idx_v[i, sl] = idx_v[i, sl] + (i * NUM_EMBEDDINGS)
            return 0
        lax.fori_loop(0, per_w // LANES, bias_body, 0)

        def g0_copy(c, b):
            return pltpu.make_async_copy(
                tab_hbm.at[idx_v.at[0, pl.ds(c * R, R)]], o_v.at[b],
                g0sem.at[b])

        def add_copy(i, c, b):
            return pltpu.make_async_copy(
                tab_hbm.at[idx_v.at[i, pl.ds(c * R, R)]], o_v.at[b],
                gasem.at[b])

        def out_copy(c, b):
            return pltpu.make_async_copy(
                o_v.at[b], out_hbm.at[pl.ds(w_base + c * R, R)], osem.at[b])

        def body(ci, _):
            b0 = lax.rem(ci, NB)
            @pl.when(ci < n_chunks)
            def _a():
                @pl.when(ci >= NB)
                def _wait_store():
                    out_copy(ci - NB, b0).wait()
                g0_copy(ci, b0).start()

            @pl.when(jnp.logical_and(ci >= 1, ci <= n_chunks))
            def _b():
                b1 = lax.rem(ci + (NB - 1), NB)
                g0_copy(ci - 1, b1).wait()
                for i in range(1, NUM_INPUTS):
                    add_copy(i, ci - 1, b1).start(add=True)

            @pl.when(ci >= 2)
            def _c():
                b2 = lax.rem(ci + (NB - 2), NB)
                for i in range(1, NUM_INPUTS):
                    add_copy(i, ci - 2, b2).wait()
                out_copy(ci - 2, b2).start()
            return 0

        lax.fori_loop(0, n_chunks + 2, body, 0)

        for c in range(n_chunks - NB, n_chunks):
            out_copy(c, c % NB).wait()

    return mimo


def kernel(x, tables):
    num, vocab, dim = tables.shape
    seq = x.shape[-1]
    batch = x.shape[0] // num
    rows = batch * seq
    xf = x.reshape(num, rows)
    tf = tables.reshape(num * vocab, dim)
    out = _build(rows)(xf, tf)
    return out.reshape(batch, seq, dim)


# speedup vs baseline: 9.2379x; 1.0012x over previous
import functools

import jax
import jax.numpy as jnp
from jax import lax
from jax.experimental import pallas as pl
from jax.experimental.pallas import tpu as pltpu
from jax.experimental.pallas import tpu_sc as plsc

NUM_INPUTS = 4
NUM_EMBEDDINGS = 100000
DIM = 64
LANES = 16
NUM_CORES = 2
NUM_SUBCORES = 16
NW = NUM_CORES * NUM_SUBCORES
R = 128
LAG = 2
NB = 6


@functools.lru_cache(maxsize=None)
def _build(rows: int):
    assert rows % (NW * R) == 0
    per_w = rows // NW
    n_chunks = per_w // R
    mesh = plsc.VectorSubcoreMesh(
        core_axis_name="c", subcore_axis_name="s",
        num_cores=NUM_CORES, num_subcores=NUM_SUBCORES)

    @functools.partial(
        pl.kernel,
        out_type=jax.ShapeDtypeStruct((rows, DIM), jnp.float32),
        mesh=mesh,
        scratch_types=[
            pltpu.VMEM((NUM_INPUTS, per_w), jnp.int32),
            pltpu.VMEM((NB, R, DIM), jnp.float32),
            pltpu.SemaphoreType.DMA,
            pltpu.SemaphoreType.DMA((NB,)),
            pltpu.SemaphoreType.DMA((NB,)),
            pltpu.SemaphoreType.DMA((NB,)),
        ],
        compiler_params=pltpu.CompilerParams(use_tc_tiling_on_sc=False),
    )
    def mimo(xf_hbm, tab_hbm, out_hbm, idx_v, o_v, isem, g0sem, gasem, osem):
        wid = lax.axis_index("s") * NUM_CORES + lax.axis_index("c")
        w_base = wid * per_w

        pltpu.async_copy(
            xf_hbm.at[:, pl.ds(w_base, per_w)], idx_v, isem).wait()

        def bias_body(j, _):
            sl = pl.ds(j * LANES, LANES)
            for i in range(1, NUM_INPUTS):
                idx_v[i, sl] = idx_v[i, sl] + (i * NUM_EMBEDDINGS)
            return 0
        lax.fori_loop(0, per_w // LANES, bias_body, 0)

        def g0_copy(c, b):
            return pltpu.make_async_copy(
                tab_hbm.at[idx_v.at[0, pl.ds(c * R, R)]], o_v.at[b],
                g0sem.at[b])

        def add_copy(i, c, b):
            return pltpu.make_async_copy(
                tab_hbm.at[idx_v.at[i, pl.ds(c * R, R)]], o_v.at[b],
                gasem.at[b])

        def out_copy(c, b):
            return pltpu.make_async_copy(
                o_v.at[b], out_hbm.at[pl.ds(w_base + c * R, R)], osem.at[b])

        def body(ci, _):
            b0 = lax.rem(ci, NB)
            @pl.when(ci < n_chunks)
            def _a():
                @pl.when(ci >= NB)
                def _wait_store():
                    out_copy(ci - NB, b0).wait()
                g0_copy(ci, b0).start()

            @pl.when(jnp.logical_and(ci >= LAG, ci < n_chunks + LAG))
            def _b():
                b1 = lax.rem(ci + (NB - LAG), NB)
                g0_copy(ci - LAG, b1).wait()
                for i in range(1, NUM_INPUTS):
                    add_copy(i, ci - LAG, b1).start(add=True)

            @pl.when(ci >= 2 * LAG)
            def _c():
                b2 = lax.rem(ci + (NB - 2 * LAG), NB)
                for i in range(1, NUM_INPUTS):
                    add_copy(i, ci - 2 * LAG, b2).wait()
                out_copy(ci - 2 * LAG, b2).start()
            return 0

        lax.fori_loop(0, n_chunks + 2 * LAG, body, 0)

        for c in range(n_chunks - NB, n_chunks):
            out_copy(c, c % NB).wait()

    return mimo


def kernel(x, tables):
    num, vocab, dim = tables.shape
    seq = x.shape[-1]
    batch = x.shape[0] // num
    rows = batch * seq
    xf = x.reshape(num, rows)
    tf = tables.reshape(num * vocab, dim)
    out = _build(rows)(xf, tf)
    return out.reshape(batch, seq, dim)
